# baseline (speedup 1.0000x reference)
import functools

import jax
import jax.numpy as jnp
from jax import lax
from jax.experimental import pallas as pl
from jax.experimental.pallas import tpu as pltpu
from jax.experimental.pallas import tpu_sc as plsc

NC, NS = 2, 16
NW = NC * NS
BATCH = 4096
SEQ = 200
D = 32
ROWS_W = BATCH // NW
ROWS = 4
CHUNK = ROWS * SEQ
NCHUNK = ROWS_W // ROWS


def _body(idx_hbm, word_hbm, pos_hbm, out_hbm, idx_v, rows_v, pos_v, sem):
    wid = lax.axis_index("s") * NC + lax.axis_index("c")
    pltpu.sync_copy(pos_hbm, pos_v)
    row_w = wid * ROWS_W

    def chunk_body(c, carry):
        row0 = row_w + c * ROWS
        for r in range(ROWS):
            pltpu.sync_copy(idx_hbm.at[row0 + r],
                            idx_v.at[pl.ds(r * SEQ, SEQ)])
        pltpu.async_copy(word_hbm.at[idx_v], rows_v, sem).wait()

        def s_body(s, carry2):
            p0 = pos_v[s, 0:16]
            p1 = pos_v[s, 16:32]
            for r in range(ROWS):
                t = r * SEQ + s
                rows_v[t, 0:16] += p0
                rows_v[t, 16:32] += p1
            return carry2

        lax.fori_loop(0, SEQ, s_body, 0)
        for r in range(ROWS):
            pltpu.sync_copy(rows_v.at[pl.ds(r * SEQ, SEQ)],
                            out_hbm.at[row0 + r])
        return carry

    lax.fori_loop(0, NCHUNK, chunk_body, 0)


def kernel(inputs, word_table, pos_table):
    mesh = plsc.VectorSubcoreMesh(core_axis_name="c", subcore_axis_name="s")
    k = pl.kernel(
        _body,
        out_type=jax.ShapeDtypeStruct((BATCH, SEQ, D), jnp.float32),
        mesh=mesh,
        scratch_types=[
            pltpu.VMEM((CHUNK,), jnp.int32),
            pltpu.VMEM((CHUNK, D), jnp.float32),
            pltpu.VMEM((SEQ, D), jnp.float32),
            pltpu.SemaphoreType.DMA,
        ],
        compiler_params=pltpu.CompilerParams(use_tc_tiling_on_sc=False),
    )
    return k(inputs.astype(jnp.int32), word_table, pos_table)
